# baseline (speedup 1.0000x reference)
import jax
import jax.numpy as jnp
from jax.experimental import pallas as pl
from jax.experimental.pallas import tpu as pltpu

_Z = 0.3


def _matmul_kernel(x_ref, w_ref, out_ref):
    x = x_ref[...]
    xhi = x.astype(jnp.bfloat16)
    r1 = x - xhi.astype(jnp.float32)
    xlo = r1.astype(jnp.bfloat16)
    r2 = r1 - xlo.astype(jnp.float32)
    xll = r2.astype(jnp.bfloat16)
    lhs = jnp.concatenate([xhi, xlo, xll], axis=1)
    out_ref[...] = jax.lax.dot_general(
        lhs, w_ref[...], (((1,), (0,)), ((), ())),
        preferred_element_type=jnp.float32)


def _finish_kernel(c0_ref, c1_ref, c2_ref, c3_ref, c4_ref, c5_ref,
                   bs_ref, be_ref, labs_ref, labe_ref, am_ref,
                   ls_ref, le_ref, ps_ref, pe_ref, ms_ref, me_ref,
                   loss_ref):
    n_total = c0_ref.shape[0] * c0_ref.shape[1]
    ls = c0_ref[...] + c2_ref[...] + c4_ref[...] + bs_ref[0, 0]
    le = c1_ref[...] + c3_ref[...] + c5_ref[...] + be_ref[0, 0]
    ls_ref[...] = ls
    le_ref[...] = le
    labs = labs_ref[...]
    labe = labe_ref[...]

    weight = (labs >= 0).astype(jnp.float32)

    def bce_sum(lg, tgt):
        per = (jnp.maximum(lg, 0.0) - lg * tgt
               + jnp.log1p(jnp.exp(-jnp.abs(lg))))
        return jnp.sum(weight * per)

    loss = (bce_sum(ls, labs.astype(jnp.float32))
            + bce_sum(le, labe.astype(jnp.float32))) / n_total
    loss_ref[...] = loss.reshape(1, 1)

    ps = jax.nn.sigmoid(ls) * weight
    pe = jax.nn.sigmoid(le) * weight
    ps_ref[...] = ps
    pe_ref[...] = pe

    ml = jnp.sum(am_ref[...], axis=1, keepdims=True) - 2
    length = (ml.astype(jnp.float32) * _Z).astype(jnp.int32)
    length = jnp.where(length < 5, 5, length)
    length = jnp.minimum(length, ml * ml)

    def kth_mask(p):
        xi = jax.lax.bitcast_convert_type(p, jnp.int32)
        t = jnp.zeros_like(length)
        for b in range(30, -1, -1):
            cand = t | (1 << b)
            cnt = jnp.sum((xi >= cand).astype(jnp.int32), axis=1,
                          keepdims=True)
            t = jnp.where(cnt >= length, cand, t)
        thr = jax.lax.bitcast_convert_type(t, jnp.float32)
        return (p >= thr).astype(jnp.uint8)

    ms_ref[...] = kth_mask(ps)
    me_ref[...] = kth_mask(pe)


def kernel(table, attention_mask, table_labels_S, table_labels_E,
           W_S, b_S, W_E, b_E):
    B, L, _, D = table.shape
    M = B * L * L
    R = 4096

    x = table.reshape(M, D)
    w = jnp.concatenate([W_S, W_E], axis=1)
    whi = w.astype(jnp.bfloat16)
    wr1 = w - whi.astype(jnp.float32)
    wlo = wr1.astype(jnp.bfloat16)
    wr2 = wr1 - wlo.astype(jnp.float32)
    wll = wr2.astype(jnp.bfloat16)
    w6 = jnp.concatenate([whi, wlo, wll], axis=1)
    rhs = jnp.concatenate([w6, w6, w6], axis=0)

    lg = pl.pallas_call(
        _matmul_kernel,
        grid=(M // R,),
        in_specs=[
            pl.BlockSpec((R, D), lambda i: (i, 0)),
            pl.BlockSpec((3 * D, 6), lambda i: (0, 0)),
        ],
        out_specs=pl.BlockSpec((R, 6), lambda i: (i, 0)),
        out_shape=jax.ShapeDtypeStruct((M, 6), jnp.float32),
        compiler_params=pltpu.CompilerParams(
            dimension_semantics=("parallel",)),
    )(x, rhs)

    lg3 = lg.reshape(B, L * L, 6)
    cols = [lg3[..., j] for j in range(6)]
    labs = table_labels_S.reshape(B, L * L)
    labe = table_labels_E.reshape(B, L * L)

    full = lambda s: pl.BlockSpec(s, lambda: (0,) * len(s))
    N = L * L
    ls, le, ps, pe, ms, me, loss = pl.pallas_call(
        _finish_kernel,
        in_specs=[full((B, N))] * 6 + [full((1, 1)), full((1, 1)),
                  full((B, N)), full((B, N)), full((B, L))],
        out_specs=[full((B, N))] * 4 + [full((B, N)), full((B, N)),
                   full((1, 1))],
        out_shape=[
            jax.ShapeDtypeStruct((B, N), jnp.float32),
            jax.ShapeDtypeStruct((B, N), jnp.float32),
            jax.ShapeDtypeStruct((B, N), jnp.float32),
            jax.ShapeDtypeStruct((B, N), jnp.float32),
            jax.ShapeDtypeStruct((B, N), jnp.uint8),
            jax.ShapeDtypeStruct((B, N), jnp.uint8),
            jax.ShapeDtypeStruct((1, 1), jnp.float32),
        ],
    )(*cols, b_S.reshape(1, 1), b_E.reshape(1, 1), labs, labe,
      attention_mask)

    logits_S = ls.reshape(B, L, L)
    logits_E = le.reshape(B, L, L)
    S_pred = ps.reshape(B, L, L)
    E_pred = pe.reshape(B, L, L)
    pred_S = (ms != 0).reshape(B, L, L)
    pred_E = (me != 0).reshape(B, L, L)
    return (loss[0, 0], S_pred, E_pred, logits_S, logits_E, pred_S, pred_E)


# speedup vs baseline: 2.2762x; 2.2762x over previous
import jax
import jax.numpy as jnp
from jax.experimental import pallas as pl
from jax.experimental.pallas import tpu as pltpu

_Z = 0.3


def _matmul_kernel(x_ref, w_ref, out_s_ref, out_e_ref):
    x = x_ref[...]
    xhi = x.astype(jnp.bfloat16)
    r1 = x - xhi.astype(jnp.float32)
    xlo = r1.astype(jnp.bfloat16)
    r2 = r1 - xlo.astype(jnp.float32)
    xll = r2.astype(jnp.bfloat16)
    lhs = jnp.concatenate([xhi, xlo, xll], axis=1)
    c = jax.lax.dot_general(
        lhs, w_ref[...], (((1,), (0,)), ((), ())),
        preferred_element_type=jnp.float32)
    y = c[:, 0:2] + c[:, 2:4] + c[:, 4:6]
    out_s_ref[...] = y[:, 0].reshape(1, 1, -1)
    out_e_ref[...] = y[:, 1].reshape(1, 1, -1)


def _finish_kernel(ls0_ref, le0_ref, bs_ref, be_ref, labs_ref, labe_ref,
                   am_ref, ls_ref, le_ref, ps_ref, pe_ref, ms_ref, me_ref,
                   loss_ref):
    n_total = ls0_ref.shape[0] * ls0_ref.shape[1]
    ls = ls0_ref[...] + bs_ref[0, 0]
    le = le0_ref[...] + be_ref[0, 0]
    ls_ref[...] = ls
    le_ref[...] = le
    labs = labs_ref[...]
    labe = labe_ref[...]

    weight = (labs >= 0).astype(jnp.float32)

    def bce_sum(lg, tgt):
        per = (jnp.maximum(lg, 0.0) - lg * tgt
               + jnp.log1p(jnp.exp(-jnp.abs(lg))))
        return jnp.sum(weight * per)

    loss = (bce_sum(ls, labs.astype(jnp.float32))
            + bce_sum(le, labe.astype(jnp.float32))) / n_total
    loss_ref[...] = loss.reshape(1, 1)

    ps = jax.nn.sigmoid(ls) * weight
    pe = jax.nn.sigmoid(le) * weight
    ps_ref[...] = ps
    pe_ref[...] = pe

    ml = jnp.sum(am_ref[...], axis=1, keepdims=True) - 2
    length = (ml.astype(jnp.float32) * _Z).astype(jnp.int32)
    length = jnp.where(length < 5, 5, length)
    length = jnp.minimum(length, ml * ml)

    def kth_mask(p):
        xi = jax.lax.bitcast_convert_type(p, jnp.int32)
        t = jnp.zeros_like(length)
        for b in range(30, -1, -1):
            cand = t | (1 << b)
            cnt = jnp.sum((xi >= cand).astype(jnp.int32), axis=1,
                          keepdims=True)
            t = jnp.where(cnt >= length, cand, t)
        thr = jax.lax.bitcast_convert_type(t, jnp.float32)
        return (p >= thr).astype(jnp.uint8)

    ms_ref[...] = kth_mask(ps)
    me_ref[...] = kth_mask(pe)


def kernel(table, attention_mask, table_labels_S, table_labels_E,
           W_S, b_S, W_E, b_E):
    B, L, _, D = table.shape
    M = B * L * L
    R = 4096

    x = table.reshape(M, D)
    w = jnp.concatenate([W_S, W_E], axis=1)
    whi = w.astype(jnp.bfloat16)
    wr1 = w - whi.astype(jnp.float32)
    wlo = wr1.astype(jnp.bfloat16)
    wr2 = wr1 - wlo.astype(jnp.float32)
    wll = wr2.astype(jnp.bfloat16)
    w6 = jnp.concatenate([whi, wlo, wll], axis=1)
    rhs = jnp.concatenate([w6, w6, w6], axis=0)

    N = L * L
    nblk = M // R
    ls3, le3 = pl.pallas_call(
        _matmul_kernel,
        grid=(nblk,),
        in_specs=[
            pl.BlockSpec((R, D), lambda i: (i, 0)),
            pl.BlockSpec((3 * D, 6), lambda i: (0, 0)),
        ],
        out_specs=[
            pl.BlockSpec((1, 1, R), lambda i: (i, 0, 0)),
            pl.BlockSpec((1, 1, R), lambda i: (i, 0, 0)),
        ],
        out_shape=[
            jax.ShapeDtypeStruct((nblk, 1, R), jnp.float32),
            jax.ShapeDtypeStruct((nblk, 1, R), jnp.float32),
        ],
        compiler_params=pltpu.CompilerParams(
            dimension_semantics=("parallel",)),
    )(x, rhs)
    ls = ls3.reshape(B, N)
    le = le3.reshape(B, N)

    labs = table_labels_S.reshape(B, L * L)
    labe = table_labels_E.reshape(B, L * L)

    full = lambda s: pl.BlockSpec(s, lambda: (0,) * len(s))
    lsb, leb, ps, pe, ms, me, loss = pl.pallas_call(
        _finish_kernel,
        in_specs=[full((B, N)), full((B, N)), full((1, 1)), full((1, 1)),
                  full((B, N)), full((B, N)), full((B, L))],
        out_specs=[full((B, N))] * 4 + [full((B, N)), full((B, N)),
                   full((1, 1))],
        out_shape=[
            jax.ShapeDtypeStruct((B, N), jnp.float32),
            jax.ShapeDtypeStruct((B, N), jnp.float32),
            jax.ShapeDtypeStruct((B, N), jnp.float32),
            jax.ShapeDtypeStruct((B, N), jnp.float32),
            jax.ShapeDtypeStruct((B, N), jnp.uint8),
            jax.ShapeDtypeStruct((B, N), jnp.uint8),
            jax.ShapeDtypeStruct((1, 1), jnp.float32),
        ],
    )(ls, le, b_S.reshape(1, 1), b_E.reshape(1, 1), labs, labe,
      attention_mask)

    logits_S = lsb.reshape(B, L, L)
    logits_E = leb.reshape(B, L, L)
    S_pred = ps.reshape(B, L, L)
    E_pred = pe.reshape(B, L, L)
    pred_S = (ms != 0).reshape(B, L, L)
    pred_E = (me != 0).reshape(B, L, L)
    return (loss[0, 0], S_pred, E_pred, logits_S, logits_E, pred_S, pred_E)


# speedup vs baseline: 3.7336x; 1.6403x over previous
import jax
import jax.numpy as jnp
from jax.experimental import pallas as pl
from jax.experimental.pallas import tpu as pltpu

_Z = 0.3


def _matmul_kernel(x_ref, w_ref, out_s_ref, out_e_ref):
    x = x_ref[...]
    xb = x.astype(jnp.bfloat16)
    y = jax.lax.dot_general(
        xb, w_ref[...], (((1,), (0,)), ((), ())),
        preferred_element_type=jnp.float32)
    out_s_ref[...] = y[:, 0].reshape(1, 1, -1)
    out_e_ref[...] = y[:, 1].reshape(1, 1, -1)


def _finish_kernel(ls0_ref, le0_ref, bs_ref, be_ref, labs_ref, labe_ref,
                   am_ref, ls_ref, le_ref, ps_ref, pe_ref, ms_ref, me_ref,
                   loss_ref):
    n_total = ls0_ref.shape[0] * ls0_ref.shape[1]
    ls = ls0_ref[...] + bs_ref[0, 0]
    le = le0_ref[...] + be_ref[0, 0]
    ls_ref[...] = ls
    le_ref[...] = le
    labs = labs_ref[...]
    labe = labe_ref[...]

    weight = (labs >= 0).astype(jnp.float32)

    def bce_sum(lg, tgt):
        per = (jnp.maximum(lg, 0.0) - lg * tgt
               + jnp.log1p(jnp.exp(-jnp.abs(lg))))
        return jnp.sum(weight * per)

    loss = (bce_sum(ls, labs.astype(jnp.float32))
            + bce_sum(le, labe.astype(jnp.float32))) / n_total
    loss_ref[...] = loss.reshape(1, 1)

    ps = jax.nn.sigmoid(ls) * weight
    pe = jax.nn.sigmoid(le) * weight
    ps_ref[...] = ps
    pe_ref[...] = pe

    ml = jnp.sum(am_ref[...], axis=1, keepdims=True) - 2
    length = (ml.astype(jnp.float32) * _Z).astype(jnp.int32)
    length = jnp.where(length < 5, 5, length)
    length = jnp.minimum(length, ml * ml)

    def kth_mask(p):
        xi = jax.lax.bitcast_convert_type(p, jnp.int32)
        t = jnp.zeros_like(length)
        for b in range(30, -1, -1):
            cand = t | (1 << b)
            cnt = jnp.sum((xi >= cand).astype(jnp.int32), axis=1,
                          keepdims=True)
            t = jnp.where(cnt >= length, cand, t)
        thr = jax.lax.bitcast_convert_type(t, jnp.float32)
        return (p >= thr).astype(jnp.uint8)

    ms_ref[...] = kth_mask(ps)
    me_ref[...] = kth_mask(pe)


def kernel(table, attention_mask, table_labels_S, table_labels_E,
           W_S, b_S, W_E, b_E):
    B, L, _, D = table.shape
    M = B * L * L
    R = 4096

    x = table.reshape(M, D)
    w = jnp.concatenate([W_S, W_E], axis=1)
    rhs = w.astype(jnp.bfloat16)

    N = L * L
    nblk = M // R
    ls3, le3 = pl.pallas_call(
        _matmul_kernel,
        grid=(nblk,),
        in_specs=[
            pl.BlockSpec((R, D), lambda i: (i, 0)),
            pl.BlockSpec((D, 2), lambda i: (0, 0)),
        ],
        out_specs=[
            pl.BlockSpec((1, 1, R), lambda i: (i, 0, 0)),
            pl.BlockSpec((1, 1, R), lambda i: (i, 0, 0)),
        ],
        out_shape=[
            jax.ShapeDtypeStruct((nblk, 1, R), jnp.float32),
            jax.ShapeDtypeStruct((nblk, 1, R), jnp.float32),
        ],
        compiler_params=pltpu.CompilerParams(
            dimension_semantics=("parallel",)),
    )(x, rhs)
    ls = ls3.reshape(B, N)
    le = le3.reshape(B, N)

    labs = table_labels_S.reshape(B, L * L)
    labe = table_labels_E.reshape(B, L * L)

    full = lambda s: pl.BlockSpec(s, lambda: (0,) * len(s))
    lsb, leb, ps, pe, ms, me, loss = pl.pallas_call(
        _finish_kernel,
        in_specs=[full((B, N)), full((B, N)), full((1, 1)), full((1, 1)),
                  full((B, N)), full((B, N)), full((B, L))],
        out_specs=[full((B, N))] * 4 + [full((B, N)), full((B, N)),
                   full((1, 1))],
        out_shape=[
            jax.ShapeDtypeStruct((B, N), jnp.float32),
            jax.ShapeDtypeStruct((B, N), jnp.float32),
            jax.ShapeDtypeStruct((B, N), jnp.float32),
            jax.ShapeDtypeStruct((B, N), jnp.float32),
            jax.ShapeDtypeStruct((B, N), jnp.uint8),
            jax.ShapeDtypeStruct((B, N), jnp.uint8),
            jax.ShapeDtypeStruct((1, 1), jnp.float32),
        ],
    )(ls, le, b_S.reshape(1, 1), b_E.reshape(1, 1), labs, labe,
      attention_mask)

    logits_S = lsb.reshape(B, L, L)
    logits_E = leb.reshape(B, L, L)
    S_pred = ps.reshape(B, L, L)
    E_pred = pe.reshape(B, L, L)
    pred_S = (ms != 0).reshape(B, L, L)
    pred_E = (me != 0).reshape(B, L, L)
    return (loss[0, 0], S_pred, E_pred, logits_S, logits_E, pred_S, pred_E)
